# baseline (device time: 72220 ns/iter reference)
import jax
import jax.numpy as jnp
from jax import lax
from jax.experimental import pallas as pl
from jax.experimental.pallas import tpu as pltpu


def kernel(dy, W):
    m, k = dy.shape
    d = W.shape[0]

    def body(dy_ref, w_ref, out_ref, comm_ref, send_sem, recv_sem):
        my_x = lax.axis_index("x")
        my_y = lax.axis_index("y")
        my_z = lax.axis_index("z")
        partner = (my_x, my_y, 1 - my_z)

        barrier_sem = pltpu.get_barrier_semaphore()
        pl.semaphore_signal(
            barrier_sem, inc=1, device_id=partner,
            device_id_type=pl.DeviceIdType.MESH,
        )
        pl.semaphore_wait(barrier_sem, 1)

        out_ref[...] = lax.dot_general(
            dy_ref[...], w_ref[...],
            dimension_numbers=(((1,), (1,)), ((), ())),
            preferred_element_type=jnp.float32,
        )

        rdma = pltpu.make_async_remote_copy(
            src_ref=out_ref,
            dst_ref=comm_ref,
            send_sem=send_sem,
            recv_sem=recv_sem,
            device_id=partner,
            device_id_type=pl.DeviceIdType.MESH,
        )
        rdma.start()
        rdma.wait()
        out_ref[...] += comm_ref[...]

    return pl.pallas_call(
        body,
        out_shape=jax.ShapeDtypeStruct((m, d), jnp.float32),
        in_specs=[
            pl.BlockSpec(memory_space=pltpu.VMEM),
            pl.BlockSpec(memory_space=pltpu.VMEM),
        ],
        out_specs=pl.BlockSpec(memory_space=pltpu.VMEM),
        scratch_shapes=[
            pltpu.VMEM((m, d), jnp.float32),
            pltpu.SemaphoreType.DMA,
            pltpu.SemaphoreType.DMA,
        ],
        compiler_params=pltpu.CompilerParams(collective_id=0),
    )(dy, W)


# device time: 46266 ns/iter; 1.5610x vs baseline; 1.5610x over previous
import jax
import jax.numpy as jnp
from jax import lax
from jax.experimental import pallas as pl
from jax.experimental.pallas import tpu as pltpu


def kernel(dy, W):
    m, k = dy.shape
    d = W.shape[0]
    qm = m // 4

    def body(dy_ref, w_ref, out_ref, sbuf, rbuf, send_sems, recv_sems):
        my_x = lax.axis_index("x")
        my_y = lax.axis_index("y")
        my_z = lax.axis_index("z")
        q = 2 * my_x + my_y
        row = q * qm
        rowd = (3 - q) * qm
        z_partner = (my_x, my_y, 1 - my_z)
        x_nbr = (1 - my_x, my_y, my_z)
        y_nbr = (my_x, 1 - my_y, my_z)

        barrier_sem = pltpu.get_barrier_semaphore()
        for nbr in (z_partner, x_nbr, y_nbr):
            pl.semaphore_signal(
                barrier_sem, inc=1, device_id=nbr,
                device_id_type=pl.DeviceIdType.MESH,
            )
        pl.semaphore_wait(barrier_sem, 3)

        dims = (((1,), (1,)), ((), ()))

        sbuf[0] = lax.dot_general(
            dy_ref[pl.ds(row, qm), :], w_ref[...],
            dimension_numbers=dims, preferred_element_type=jnp.float32,
        )
        rdma_z0 = pltpu.make_async_remote_copy(
            src_ref=sbuf.at[0], dst_ref=rbuf.at[0],
            send_sem=send_sems.at[0], recv_sem=recv_sems.at[0],
            device_id=z_partner, device_id_type=pl.DeviceIdType.MESH,
        )
        rdma_z0.start()

        sbuf[1] = lax.dot_general(
            dy_ref[pl.ds(rowd, qm), :], w_ref[...],
            dimension_numbers=dims, preferred_element_type=jnp.float32,
        )
        rdma_z1 = pltpu.make_async_remote_copy(
            src_ref=sbuf.at[1], dst_ref=rbuf.at[1],
            send_sem=send_sems.at[1], recv_sem=recv_sems.at[1],
            device_id=z_partner, device_id_type=pl.DeviceIdType.MESH,
        )
        rdma_z1.start()

        rdma_z0.wait_recv()
        out_ref[pl.ds(row, qm), :] = sbuf[0] + rbuf[0]
        rdma_x = pltpu.make_async_remote_copy(
            src_ref=out_ref.at[pl.ds(row, qm), :],
            dst_ref=out_ref.at[pl.ds(row, qm), :],
            send_sem=send_sems.at[2], recv_sem=recv_sems.at[2],
            device_id=x_nbr, device_id_type=pl.DeviceIdType.MESH,
        )
        rdma_x.start()
        rdma_y = pltpu.make_async_remote_copy(
            src_ref=out_ref.at[pl.ds(row, qm), :],
            dst_ref=out_ref.at[pl.ds(row, qm), :],
            send_sem=send_sems.at[3], recv_sem=recv_sems.at[3],
            device_id=y_nbr, device_id_type=pl.DeviceIdType.MESH,
        )
        rdma_y.start()

        rdma_z1.wait_recv()
        out_ref[pl.ds(rowd, qm), :] = sbuf[1] + rbuf[1]

        rdma_x.wait_recv()
        rdma_y.wait_recv()
        rdma_z0.wait_send()
        rdma_z1.wait_send()
        rdma_x.wait_send()
        rdma_y.wait_send()

    return pl.pallas_call(
        body,
        out_shape=jax.ShapeDtypeStruct((m, d), jnp.float32),
        in_specs=[
            pl.BlockSpec(memory_space=pltpu.VMEM),
            pl.BlockSpec(memory_space=pltpu.VMEM),
        ],
        out_specs=pl.BlockSpec(memory_space=pltpu.VMEM),
        scratch_shapes=[
            pltpu.VMEM((2, qm, d), jnp.float32),
            pltpu.VMEM((2, qm, d), jnp.float32),
            pltpu.SemaphoreType.DMA((4,)),
            pltpu.SemaphoreType.DMA((4,)),
        ],
        compiler_params=pltpu.CompilerParams(collective_id=0),
    )(dy, W)


# device time: 44520 ns/iter; 1.6222x vs baseline; 1.0392x over previous
import jax
import jax.numpy as jnp
from jax import lax
from jax.experimental import pallas as pl
from jax.experimental.pallas import tpu as pltpu

C = 2


def kernel(dy, W):
    m, k = dy.shape
    d = W.shape[0]
    qm = m // 4
    ch = qm // C

    def body(dy_ref, w_ref, out_ref, sbuf, rbuf,
             zsend, zrecv, xsend, xrecv, ysend, yrecv):
        my_x = lax.axis_index("x")
        my_y = lax.axis_index("y")
        my_z = lax.axis_index("z")
        q = 2 * my_x + my_y
        row = q * qm
        rowd = (3 - q) * qm
        z_partner = (my_x, my_y, 1 - my_z)
        x_nbr = (1 - my_x, my_y, my_z)
        y_nbr = (my_x, 1 - my_y, my_z)

        barrier_sem = pltpu.get_barrier_semaphore()
        for nbr in (z_partner, x_nbr, y_nbr):
            pl.semaphore_signal(
                barrier_sem, inc=1, device_id=nbr,
                device_id_type=pl.DeviceIdType.MESH,
            )
        pl.semaphore_wait(barrier_sem, 3)

        dims = (((1,), (1,)), ((), ()))

        def z_rdma(slot):
            return pltpu.make_async_remote_copy(
                src_ref=sbuf.at[slot], dst_ref=rbuf.at[slot],
                send_sem=zsend.at[slot], recv_sem=zrecv.at[slot],
                device_id=z_partner, device_id_type=pl.DeviceIdType.MESH,
            )

        z_rdmas = []
        for s in range(2 * C):
            base = row if s < C else rowd
            r0 = base + (s % C) * ch
            sbuf[s] = lax.dot_general(
                dy_ref[pl.ds(r0, ch), :], w_ref[...],
                dimension_numbers=dims, preferred_element_type=jnp.float32,
            )
            rd = z_rdma(s)
            rd.start()
            z_rdmas.append(rd)

        xy_rdmas = []
        for c in range(C):
            z_rdmas[c].wait_recv()
            r0 = row + c * ch
            out_ref[pl.ds(r0, ch), :] = sbuf[c] + rbuf[c]
            for nbr, ssem, rsem in ((x_nbr, xsend, xrecv),
                                    (y_nbr, ysend, yrecv)):
                rd = pltpu.make_async_remote_copy(
                    src_ref=out_ref.at[pl.ds(r0, ch), :],
                    dst_ref=out_ref.at[pl.ds(r0, ch), :],
                    send_sem=ssem.at[c], recv_sem=rsem.at[c],
                    device_id=nbr, device_id_type=pl.DeviceIdType.MESH,
                )
                rd.start()
                xy_rdmas.append(rd)

        for c in range(C):
            z_rdmas[C + c].wait_recv()
            out_ref[pl.ds(rowd + c * ch, ch), :] = sbuf[C + c] + rbuf[C + c]

        for rd in xy_rdmas:
            rd.wait_recv()
        for rd in z_rdmas + xy_rdmas:
            rd.wait_send()

    return pl.pallas_call(
        body,
        out_shape=jax.ShapeDtypeStruct((m, d), jnp.float32),
        in_specs=[
            pl.BlockSpec(memory_space=pltpu.VMEM),
            pl.BlockSpec(memory_space=pltpu.VMEM),
        ],
        out_specs=pl.BlockSpec(memory_space=pltpu.VMEM),
        scratch_shapes=[
            pltpu.VMEM((2 * C, ch, d), jnp.float32),
            pltpu.VMEM((2 * C, ch, d), jnp.float32),
            pltpu.SemaphoreType.DMA((2 * C,)),
            pltpu.SemaphoreType.DMA((2 * C,)),
            pltpu.SemaphoreType.DMA((C,)),
            pltpu.SemaphoreType.DMA((C,)),
            pltpu.SemaphoreType.DMA((C,)),
            pltpu.SemaphoreType.DMA((C,)),
        ],
        compiler_params=pltpu.CompilerParams(collective_id=0),
    )(dy, W)


# device time: 35027 ns/iter; 2.0618x vs baseline; 1.2710x over previous
import jax
import jax.numpy as jnp
from jax import lax
from jax.experimental import pallas as pl
from jax.experimental.pallas import tpu as pltpu


def kernel(dy, W):
    m, k = dy.shape
    d = W.shape[0]
    qm = m // 4

    def body(dy_hbm, w_ref, out_ref, dyf, dyb, wb, sbuf, rbuf, obuf,
             copy_sems, send_sems, recv_sems):
        my_x = lax.axis_index("x")
        my_y = lax.axis_index("y")
        my_z = lax.axis_index("z")
        q = 2 * my_x + my_y
        row = q * qm
        rowd = (3 - q) * qm
        z_partner = (my_x, my_y, 1 - my_z)
        x_nbr = (1 - my_x, my_y, my_z)
        y_nbr = (my_x, 1 - my_y, my_z)

        cp_own = pltpu.make_async_copy(
            dy_hbm.at[pl.ds(row, qm), :], dyf.at[0], copy_sems.at[0])
        cp_diag = pltpu.make_async_copy(
            dy_hbm.at[pl.ds(rowd, qm), :], dyf.at[1], copy_sems.at[1])
        cp_own.start()
        cp_diag.start()

        barrier_sem = pltpu.get_barrier_semaphore()
        for nbr in (z_partner, x_nbr, y_nbr):
            pl.semaphore_signal(
                barrier_sem, inc=1, device_id=nbr,
                device_id_type=pl.DeviceIdType.MESH,
            )
        pl.semaphore_wait(barrier_sem, 3)

        wb[...] = w_ref[...].astype(jnp.bfloat16)

        dims = (((1,), (1,)), ((), ()))

        def z_rdma(slot):
            return pltpu.make_async_remote_copy(
                src_ref=sbuf.at[slot], dst_ref=rbuf.at[slot],
                send_sem=send_sems.at[slot], recv_sem=recv_sems.at[slot],
                device_id=z_partner, device_id_type=pl.DeviceIdType.MESH,
            )

        cp_own.wait()
        dyb[0] = dyf[0].astype(jnp.bfloat16)
        sbuf[0] = lax.dot_general(
            dyb[0], wb[...], dimension_numbers=dims,
            preferred_element_type=jnp.float32,
        ).astype(jnp.bfloat16)
        rdma_z0 = z_rdma(0)
        rdma_z0.start()

        cp_diag.wait()
        dyb[1] = dyf[1].astype(jnp.bfloat16)
        sbuf[1] = lax.dot_general(
            dyb[1], wb[...], dimension_numbers=dims,
            preferred_element_type=jnp.float32,
        ).astype(jnp.bfloat16)
        rdma_z1 = z_rdma(1)
        rdma_z1.start()

        rdma_z0.wait_recv()
        obuf[0] = sbuf[0] + rbuf[0]
        xy_rdmas = []
        for i, (nbr, ssem, rsem) in enumerate(
                ((x_nbr, send_sems.at[2], recv_sems.at[2]),
                 (y_nbr, send_sems.at[3], recv_sems.at[3]))):
            rd = pltpu.make_async_remote_copy(
                src_ref=obuf.at[0], dst_ref=obuf.at[1 + i],
                send_sem=ssem, recv_sem=rsem,
                device_id=nbr, device_id_type=pl.DeviceIdType.MESH,
            )
            rd.start()
            xy_rdmas.append(rd)
        out_ref[pl.ds(row, qm), :] = obuf[0].astype(jnp.float32)

        rdma_z1.wait_recv()
        out_ref[pl.ds(rowd, qm), :] = (sbuf[1] + rbuf[1]).astype(jnp.float32)

        xy_rdmas[0].wait_recv()
        out_ref[pl.ds((q ^ 2) * qm, qm), :] = obuf[1].astype(jnp.float32)
        xy_rdmas[1].wait_recv()
        out_ref[pl.ds((q ^ 1) * qm, qm), :] = obuf[2].astype(jnp.float32)

        rdma_z0.wait_send()
        rdma_z1.wait_send()
        for rd in xy_rdmas:
            rd.wait_send()

    return pl.pallas_call(
        body,
        out_shape=jax.ShapeDtypeStruct((m, d), jnp.float32),
        in_specs=[
            pl.BlockSpec(memory_space=pltpu.MemorySpace.HBM),
            pl.BlockSpec(memory_space=pltpu.VMEM),
        ],
        out_specs=pl.BlockSpec(memory_space=pltpu.VMEM),
        scratch_shapes=[
            pltpu.VMEM((2, qm, k), jnp.float32),
            pltpu.VMEM((2, qm, k), jnp.bfloat16),
            pltpu.VMEM((d, k), jnp.bfloat16),
            pltpu.VMEM((2, qm, d), jnp.bfloat16),
            pltpu.VMEM((2, qm, d), jnp.bfloat16),
            pltpu.VMEM((3, qm, d), jnp.bfloat16),
            pltpu.SemaphoreType.DMA((2,)),
            pltpu.SemaphoreType.DMA((4,)),
            pltpu.SemaphoreType.DMA((4,)),
        ],
        compiler_params=pltpu.CompilerParams(
            collective_id=0, vmem_limit_bytes=56 * 1024 * 1024,
        ),
    )(dy, W)


# device time: 33065 ns/iter; 2.1842x vs baseline; 1.0593x over previous
import jax
import jax.numpy as jnp
from jax import lax
from jax.experimental import pallas as pl
from jax.experimental.pallas import tpu as pltpu

KC = 4


def kernel(dy, W):
    m, k = dy.shape
    d = W.shape[0]
    qm = m // 4
    kc = k // KC

    def body(dy_hbm, w_hbm, out_ref, dyf, dyb, wf, wb, sbuf, rbuf, obuf,
             copy_sems, send_sems, recv_sems):
        my_x = lax.axis_index("x")
        my_y = lax.axis_index("y")
        my_z = lax.axis_index("z")
        q = 2 * my_x + my_y
        row = q * qm
        rowd = (3 - q) * qm
        z_partner = (my_x, my_y, 1 - my_z)
        x_nbr = (1 - my_x, my_y, my_z)
        y_nbr = (my_x, 1 - my_y, my_z)

        cp_own = pltpu.make_async_copy(
            dy_hbm.at[pl.ds(row, qm), :], dyf.at[0], copy_sems.at[0])
        cp_own.start()
        cp_w = []
        for c in range(KC):
            cp = pltpu.make_async_copy(
                w_hbm.at[:, pl.ds(c * kc, kc)],
                wf.at[:, pl.ds(c * kc, kc)],
                copy_sems.at[2 + c])
            cp.start()
            cp_w.append(cp)
        cp_diag = pltpu.make_async_copy(
            dy_hbm.at[pl.ds(rowd, qm), :], dyf.at[1], copy_sems.at[1])
        cp_diag.start()

        barrier_sem = pltpu.get_barrier_semaphore()
        for nbr in (z_partner, x_nbr, y_nbr):
            pl.semaphore_signal(
                barrier_sem, inc=1, device_id=nbr,
                device_id_type=pl.DeviceIdType.MESH,
            )
        pl.semaphore_wait(barrier_sem, 3)

        dims = (((1,), (1,)), ((), ()))

        def z_rdma(slot):
            return pltpu.make_async_remote_copy(
                src_ref=sbuf.at[slot], dst_ref=rbuf.at[slot],
                send_sem=send_sems.at[slot], recv_sem=recv_sems.at[slot],
                device_id=z_partner, device_id_type=pl.DeviceIdType.MESH,
            )

        cp_own.wait()
        dyb[0] = dyf[0].astype(jnp.bfloat16)
        acc = None
        for c in range(KC):
            cp_w[c].wait()
            csl = pl.ds(c * kc, kc)
            wb[:, csl] = wf[:, csl].astype(jnp.bfloat16)
            p = lax.dot_general(
                dyb[0, :, csl], wb[:, csl], dimension_numbers=dims,
                preferred_element_type=jnp.float32,
            )
            acc = p if acc is None else acc + p
        sbuf[0] = acc.astype(jnp.bfloat16)
        rdma_z0 = z_rdma(0)
        rdma_z0.start()

        cp_diag.wait()
        dyb[1] = dyf[1].astype(jnp.bfloat16)
        acc2 = None
        for c in range(KC):
            csl = pl.ds(c * kc, kc)
            p = lax.dot_general(
                dyb[1, :, csl], wb[:, csl], dimension_numbers=dims,
                preferred_element_type=jnp.float32,
            )
            acc2 = p if acc2 is None else acc2 + p
        sbuf[1] = acc2.astype(jnp.bfloat16)
        rdma_z1 = z_rdma(1)
        rdma_z1.start()

        rdma_z0.wait_recv()
        obuf[0] = sbuf[0] + rbuf[0]
        xy_rdmas = []
        for i, (nbr, ssem, rsem) in enumerate(
                ((x_nbr, send_sems.at[2], recv_sems.at[2]),
                 (y_nbr, send_sems.at[3], recv_sems.at[3]))):
            rd = pltpu.make_async_remote_copy(
                src_ref=obuf.at[0], dst_ref=obuf.at[1 + i],
                send_sem=ssem, recv_sem=rsem,
                device_id=nbr, device_id_type=pl.DeviceIdType.MESH,
            )
            rd.start()
            xy_rdmas.append(rd)
        out_ref[pl.ds(row, qm), :] = obuf[0].astype(jnp.float32)

        rdma_z1.wait_recv()
        out_ref[pl.ds(rowd, qm), :] = (sbuf[1] + rbuf[1]).astype(jnp.float32)

        xy_rdmas[0].wait_recv()
        out_ref[pl.ds((q ^ 2) * qm, qm), :] = obuf[1].astype(jnp.float32)
        xy_rdmas[1].wait_recv()
        out_ref[pl.ds((q ^ 1) * qm, qm), :] = obuf[2].astype(jnp.float32)

        rdma_z0.wait_send()
        rdma_z1.wait_send()
        for rd in xy_rdmas:
            rd.wait_send()

    return pl.pallas_call(
        body,
        out_shape=jax.ShapeDtypeStruct((m, d), jnp.float32),
        in_specs=[
            pl.BlockSpec(memory_space=pltpu.MemorySpace.HBM),
            pl.BlockSpec(memory_space=pltpu.MemorySpace.HBM),
        ],
        out_specs=pl.BlockSpec(memory_space=pltpu.VMEM),
        scratch_shapes=[
            pltpu.VMEM((2, qm, k), jnp.float32),
            pltpu.VMEM((2, qm, k), jnp.bfloat16),
            pltpu.VMEM((d, k), jnp.float32),
            pltpu.VMEM((d, k), jnp.bfloat16),
            pltpu.VMEM((2, qm, d), jnp.bfloat16),
            pltpu.VMEM((2, qm, d), jnp.bfloat16),
            pltpu.VMEM((3, qm, d), jnp.bfloat16),
            pltpu.SemaphoreType.DMA((6,)),
            pltpu.SemaphoreType.DMA((4,)),
            pltpu.SemaphoreType.DMA((4,)),
        ],
        compiler_params=pltpu.CompilerParams(
            collective_id=0, vmem_limit_bytes=56 * 1024 * 1024,
        ),
    )(dy, W)


# device time: 32831 ns/iter; 2.1998x vs baseline; 1.0071x over previous
import jax
import jax.numpy as jnp
from jax import lax
from jax.experimental import pallas as pl
from jax.experimental.pallas import tpu as pltpu

KC = 4
RC = 2


def kernel(dy, W):
    m, k = dy.shape
    d = W.shape[0]
    qm = m // 4
    kc = k // KC
    rc = qm // RC

    NZ = RC + 1

    def body(dy_hbm, w_hbm, out_ref, dyf, dyb, wf, wb, sbuf, rbuf, obuf,
             copy_sems, zsend, zrecv, xysend, xyrecv):
        my_x = lax.axis_index("x")
        my_y = lax.axis_index("y")
        my_z = lax.axis_index("z")
        q = 2 * my_x + my_y
        row = q * qm
        rowd = (3 - q) * qm
        z_partner = (my_x, my_y, 1 - my_z)
        x_nbr = (1 - my_x, my_y, my_z)
        y_nbr = (my_x, 1 - my_y, my_z)

        cp_own = pltpu.make_async_copy(
            dy_hbm.at[pl.ds(row, qm), :], dyf.at[0], copy_sems.at[0])
        cp_own.start()
        cp_w = []
        for c in range(KC):
            cp = pltpu.make_async_copy(
                w_hbm.at[:, pl.ds(c * kc, kc)],
                wf.at[:, pl.ds(c * kc, kc)],
                copy_sems.at[2 + c])
            cp.start()
            cp_w.append(cp)
        cp_diag = pltpu.make_async_copy(
            dy_hbm.at[pl.ds(rowd, qm), :], dyf.at[1], copy_sems.at[1])
        cp_diag.start()

        barrier_sem = pltpu.get_barrier_semaphore()
        for nbr in (z_partner, x_nbr, y_nbr):
            pl.semaphore_signal(
                barrier_sem, inc=1, device_id=nbr,
                device_id_type=pl.DeviceIdType.MESH,
            )
        pl.semaphore_wait(barrier_sem, 3)

        dims = (((1,), (1,)), ((), ()))

        def z_rdma(slot, start, size):
            return pltpu.make_async_remote_copy(
                src_ref=sbuf.at[pl.ds(start, size), :],
                dst_ref=rbuf.at[pl.ds(start, size), :],
                send_sem=zsend.at[slot], recv_sem=zrecv.at[slot],
                device_id=z_partner, device_id_type=pl.DeviceIdType.MESH,
            )

        cp_own.wait()
        dyb[0] = dyf[0].astype(jnp.bfloat16)
        accs = [None] * RC
        for c in range(KC):
            cp_w[c].wait()
            csl = pl.ds(c * kc, kc)
            wb[:, csl] = wf[:, csl].astype(jnp.bfloat16)
            for r in range(RC):
                p = lax.dot_general(
                    dyb[0, pl.ds(r * rc, rc), csl], wb[:, csl],
                    dimension_numbers=dims,
                    preferred_element_type=jnp.float32,
                )
                accs[r] = p if accs[r] is None else accs[r] + p
        z_rdmas = []
        for r in range(RC):
            sbuf[pl.ds(r * rc, rc), :] = accs[r].astype(jnp.bfloat16)
            rd = z_rdma(r, r * rc, rc)
            rd.start()
            z_rdmas.append(rd)

        cp_diag.wait()
        dyb[1] = dyf[1].astype(jnp.bfloat16)
        acc2 = None
        for c in range(KC):
            csl = pl.ds(c * kc, kc)
            p = lax.dot_general(
                dyb[1, :, csl], wb[:, csl], dimension_numbers=dims,
                preferred_element_type=jnp.float32,
            )
            acc2 = p if acc2 is None else acc2 + p
        sbuf[pl.ds(RC * rc, qm), :] = acc2.astype(jnp.bfloat16)
        rd_diag = z_rdma(RC, RC * rc, qm)
        rd_diag.start()

        xy_rdmas = []
        for r in range(RC):
            z_rdmas[r].wait_recv()
            rsl = pl.ds(r * rc, rc)
            obuf[0, rsl, :] = sbuf[rsl, :] + rbuf[rsl, :]
            for i, nbr in enumerate((x_nbr, y_nbr)):
                rd = pltpu.make_async_remote_copy(
                    src_ref=obuf.at[0, rsl, :], dst_ref=obuf.at[1 + i, rsl, :],
                    send_sem=xysend.at[2 * r + i], recv_sem=xyrecv.at[2 * r + i],
                    device_id=nbr, device_id_type=pl.DeviceIdType.MESH,
                )
                rd.start()
                xy_rdmas.append(rd)
            out_ref[pl.ds(row + r * rc, rc), :] = obuf[0, rsl, :].astype(
                jnp.float32)

        rd_diag.wait_recv()
        dsl = pl.ds(RC * rc, qm)
        out_ref[pl.ds(rowd, qm), :] = (sbuf[dsl, :] + rbuf[dsl, :]).astype(
            jnp.float32)

        for r in range(RC):
            rsl = pl.ds(r * rc, rc)
            xy_rdmas[2 * r].wait_recv()
            out_ref[pl.ds((q ^ 2) * qm + r * rc, rc), :] = obuf[
                1, rsl, :].astype(jnp.float32)
            xy_rdmas[2 * r + 1].wait_recv()
            out_ref[pl.ds((q ^ 1) * qm + r * rc, rc), :] = obuf[
                2, rsl, :].astype(jnp.float32)

        for rd in z_rdmas + [rd_diag] + xy_rdmas:
            rd.wait_send()

    return pl.pallas_call(
        body,
        out_shape=jax.ShapeDtypeStruct((m, d), jnp.float32),
        in_specs=[
            pl.BlockSpec(memory_space=pltpu.MemorySpace.HBM),
            pl.BlockSpec(memory_space=pltpu.MemorySpace.HBM),
        ],
        out_specs=pl.BlockSpec(memory_space=pltpu.VMEM),
        scratch_shapes=[
            pltpu.VMEM((2, qm, k), jnp.float32),
            pltpu.VMEM((2, qm, k), jnp.bfloat16),
            pltpu.VMEM((d, k), jnp.float32),
            pltpu.VMEM((d, k), jnp.bfloat16),
            pltpu.VMEM((2 * qm, d), jnp.bfloat16),
            pltpu.VMEM((2 * qm, d), jnp.bfloat16),
            pltpu.VMEM((3, qm, d), jnp.bfloat16),
            pltpu.SemaphoreType.DMA((6,)),
            pltpu.SemaphoreType.DMA((NZ,)),
            pltpu.SemaphoreType.DMA((NZ,)),
            pltpu.SemaphoreType.DMA((2 * RC,)),
            pltpu.SemaphoreType.DMA((2 * RC,)),
        ],
        compiler_params=pltpu.CompilerParams(
            collective_id=0, vmem_limit_bytes=56 * 1024 * 1024,
        ),
    )(dy, W)


# device time: 27789 ns/iter; 2.5989x vs baseline; 1.1814x over previous
import jax
import jax.numpy as jnp
from jax import lax
from jax.experimental import pallas as pl
from jax.experimental.pallas import tpu as pltpu

DC = 4


def kernel(dy, W):
    m, k = dy.shape
    d = W.shape[0]
    qm = m // 4
    dcs = d // DC

    def body(dy_hbm, w_hbm, out_hbm, dyf, dyb, wf, wb, sbuf, rbuf, obuf,
             stage, copy_sems, zsend, zrecv, xysend, xyrecv, out_sems):
        my_x = lax.axis_index("x")
        my_y = lax.axis_index("y")
        my_z = lax.axis_index("z")
        q = 2 * my_x + my_y
        row = q * qm
        rowd = (3 - q) * qm
        z_partner = (my_x, my_y, 1 - my_z)
        x_nbr = (1 - my_x, my_y, my_z)
        y_nbr = (my_x, 1 - my_y, my_z)

        out_dmas = []

        def store(gr, gc, value):
            rsl, csl = pl.ds(gr, qm), pl.ds(gc, dcs)
            stage[rsl, csl] = value
            cp = pltpu.make_async_copy(
                stage.at[rsl, csl], out_hbm.at[rsl, csl],
                out_sems.at[len(out_dmas)])
            cp.start()
            out_dmas.append(cp)

        cp_own = pltpu.make_async_copy(
            dy_hbm.at[pl.ds(row, qm), :], dyf.at[0], copy_sems.at[0])
        cp_own.start()
        cp_w = []
        for c in range(DC):
            cp = pltpu.make_async_copy(
                w_hbm.at[pl.ds(c * dcs, dcs), :],
                wf.at[pl.ds(c * dcs, dcs), :],
                copy_sems.at[2 + c])
            cp.start()
            cp_w.append(cp)
        cp_diag = pltpu.make_async_copy(
            dy_hbm.at[pl.ds(rowd, qm), :], dyf.at[1], copy_sems.at[1])
        cp_diag.start()

        barrier_sem = pltpu.get_barrier_semaphore()
        for nbr in (z_partner, x_nbr, y_nbr):
            pl.semaphore_signal(
                barrier_sem, inc=1, device_id=nbr,
                device_id_type=pl.DeviceIdType.MESH,
            )
        pl.semaphore_wait(barrier_sem, 3)

        dims = (((1,), (1,)), ((), ()))

        def z_rdma(slot, r0, c0):
            return pltpu.make_async_remote_copy(
                src_ref=sbuf.at[pl.ds(r0, qm), pl.ds(c0, dcs)],
                dst_ref=rbuf.at[pl.ds(r0, qm), pl.ds(c0, dcs)],
                send_sem=zsend.at[slot], recv_sem=zrecv.at[slot],
                device_id=z_partner, device_id_type=pl.DeviceIdType.MESH,
            )

        cp_own.wait()
        dyb[0] = dyf[0].astype(jnp.bfloat16)
        z_own = []
        for c in range(DC):
            cp_w[c].wait()
            wsl = pl.ds(c * dcs, dcs)
            wb[wsl, :] = wf[wsl, :].astype(jnp.bfloat16)
            p = lax.dot_general(
                dyb[0], wb[wsl, :], dimension_numbers=dims,
                preferred_element_type=jnp.float32,
            )
            sbuf[0:qm, wsl] = p.astype(jnp.bfloat16)
            rd = z_rdma(c, 0, c * dcs)
            rd.start()
            z_own.append(rd)

        cp_diag.wait()
        dyb[1] = dyf[1].astype(jnp.bfloat16)
        z_diag = []
        for c in range(DC):
            wsl = pl.ds(c * dcs, dcs)
            p = lax.dot_general(
                dyb[1], wb[wsl, :], dimension_numbers=dims,
                preferred_element_type=jnp.float32,
            )
            sbuf[qm:2 * qm, wsl] = p.astype(jnp.bfloat16)
            rd = z_rdma(DC + c, qm, c * dcs)
            rd.start()
            z_diag.append(rd)

        xy_rdmas = []
        for c in range(DC):
            z_own[c].wait_recv()
            csl = pl.ds(c * dcs, dcs)
            obuf[0:qm, csl] = sbuf[0:qm, csl] + rbuf[0:qm, csl]
            for i, nbr in enumerate((x_nbr, y_nbr)):
                rd = pltpu.make_async_remote_copy(
                    src_ref=obuf.at[pl.ds(0, qm), csl],
                    dst_ref=obuf.at[pl.ds((1 + i) * qm, qm), csl],
                    send_sem=xysend.at[2 * c + i],
                    recv_sem=xyrecv.at[2 * c + i],
                    device_id=nbr, device_id_type=pl.DeviceIdType.MESH,
                )
                rd.start()
                xy_rdmas.append(rd)
            store(row, c * dcs, obuf[0:qm, csl].astype(jnp.float32))

        for c in range(DC):
            z_diag[c].wait_recv()
            csl = pl.ds(c * dcs, dcs)
            store(rowd, c * dcs,
                  (sbuf[qm:2 * qm, csl] + rbuf[qm:2 * qm, csl]).astype(
                      jnp.float32))

        for c in range(DC):
            csl = pl.ds(c * dcs, dcs)
            xy_rdmas[2 * c].wait_recv()
            store((q ^ 2) * qm, c * dcs,
                  obuf[qm:2 * qm, csl].astype(jnp.float32))
            xy_rdmas[2 * c + 1].wait_recv()
            store((q ^ 1) * qm, c * dcs,
                  obuf[2 * qm:3 * qm, csl].astype(jnp.float32))

        for rd in z_own + z_diag + xy_rdmas:
            rd.wait_send()
        for cp in out_dmas:
            cp.wait()

    return pl.pallas_call(
        body,
        out_shape=jax.ShapeDtypeStruct((m, d), jnp.float32),
        in_specs=[
            pl.BlockSpec(memory_space=pltpu.MemorySpace.HBM),
            pl.BlockSpec(memory_space=pltpu.MemorySpace.HBM),
        ],
        out_specs=pl.BlockSpec(memory_space=pltpu.MemorySpace.HBM),
        scratch_shapes=[
            pltpu.VMEM((2, qm, k), jnp.float32),
            pltpu.VMEM((2, qm, k), jnp.bfloat16),
            pltpu.VMEM((d, k), jnp.float32),
            pltpu.VMEM((d, k), jnp.bfloat16),
            pltpu.VMEM((2 * qm, d), jnp.bfloat16),
            pltpu.VMEM((2 * qm, d), jnp.bfloat16),
            pltpu.VMEM((3 * qm, d), jnp.bfloat16),
            pltpu.VMEM((m, d), jnp.float32),
            pltpu.SemaphoreType.DMA((2 + DC,)),
            pltpu.SemaphoreType.DMA((2 * DC,)),
            pltpu.SemaphoreType.DMA((2 * DC,)),
            pltpu.SemaphoreType.DMA((2 * DC,)),
            pltpu.SemaphoreType.DMA((2 * DC,)),
            pltpu.SemaphoreType.DMA((4 * DC,)),
        ],
        compiler_params=pltpu.CompilerParams(
            collective_id=0, vmem_limit_bytes=56 * 1024 * 1024,
        ),
    )(dy, W)
